# Initial kernel scaffold; baseline (speedup 1.0000x reference)
import jax
import jax.numpy as jnp
from jax.experimental import pallas as pl


def kernel(z_e, codebook):
    raise NotImplementedError("write your pallas kernel here")


# speedup vs baseline: 1.6456x; 1.6456x over previous
import functools

import jax
import jax.numpy as jnp
from jax.experimental import pallas as pl
from jax.experimental.pallas import tpu as pltpu

TEMP = 0.01
COMMIT = 0.25


def _vq_body(nsteps, total_tokens, z_ref, ct_ref, c_ref,
             idx_ref, q_ref, ent_ref, emb_ref, com_ref,
             accp_ref, accpl_ref, accsq_ref):
    i = pl.program_id(0)
    z = z_ref[...]
    ct = ct_ref[...]
    n = ct.shape[1]
    zsq = jnp.sum(z * z, axis=1, keepdims=True)
    csq = jnp.sum(ct * ct, axis=0, keepdims=True)
    dots = jax.lax.dot_general(
        z, ct, (((1,), (0,)), ((), ())),
        preferred_element_type=jnp.float32,
        precision=jax.lax.Precision.DEFAULT)
    d = zsq - 2.0 * dots + csq

    idx = jnp.argmin(d, axis=1)
    idx_ref[...] = idx.reshape(idx_ref.shape)

    onehot = (idx[:, None] == jax.lax.broadcasted_iota(
        jnp.int32, (1, n), 1)).astype(jnp.float32)
    q = jax.lax.dot_general(
        onehot, c_ref[...], (((1,), (0,)), ((), ())),
        preferred_element_type=jnp.float32,
        precision=jax.lax.Precision.HIGHEST)
    q_ref[...] = q
    diff = q - z
    sq = jnp.sum(diff * diff, keepdims=True).reshape(1, 1)

    aff = (-d) / TEMP
    m = jnp.max(aff, axis=1, keepdims=True)
    e = jnp.exp(aff - m)
    s = jnp.sum(e, axis=1, keepdims=True)
    probs = e / s
    y = aff + 1e-5
    sh = y - jnp.max(y, axis=1, keepdims=True)
    ls = jnp.log(jnp.sum(jnp.exp(sh), axis=1, keepdims=True))
    logp = sh - ls
    plsum = jnp.sum(probs * logp, keepdims=True).reshape(1, 1)
    colsum = jnp.sum(probs, axis=0, keepdims=True)

    @pl.when(i == 0)
    def _():
        accp_ref[...] = colsum
        accpl_ref[...] = plsum
        accsq_ref[...] = sq

    @pl.when(i > 0)
    def _():
        accp_ref[...] += colsum
        accpl_ref[...] += plsum
        accsq_ref[...] += sq

    @pl.when(i == nsteps - 1)
    def _():
        tt = jnp.float32(total_tokens)
        avg_probs = accp_ref[...] / tt
        avg_entropy = -jnp.sum(avg_probs * jnp.log(avg_probs + 1e-5),
                               keepdims=True).reshape(1, 1)
        sample_entropy = -(accpl_ref[...] / tt)
        ent_ref[...] = 0.1 * (sample_entropy - avg_entropy)
        msq = accsq_ref[...] / (tt * z.shape[1])
        emb_ref[...] = msq
        com_ref[...] = COMMIT * msq


def kernel(z_e, codebook):
    codebook = jnp.asarray(codebook, dtype=jnp.float32)
    n, d = codebook.shape
    z_flat = jnp.reshape(z_e, (-1, d)).astype(jnp.float32)
    t = z_flat.shape[0]
    bt = 1024
    nsteps = t // bt
    ct = codebook.T

    out_shapes = (
        jax.ShapeDtypeStruct((nsteps, 1, bt), jnp.int32),
        jax.ShapeDtypeStruct((t, d), jnp.float32),
        jax.ShapeDtypeStruct((1, 1), jnp.float32),
        jax.ShapeDtypeStruct((1, 1), jnp.float32),
        jax.ShapeDtypeStruct((1, 1), jnp.float32),
    )
    idx3, q, ent, emb, com = pl.pallas_call(
        functools.partial(_vq_body, nsteps, t),
        grid=(nsteps,),
        in_specs=[
            pl.BlockSpec((bt, d), lambda i: (i, 0)),
            pl.BlockSpec((d, n), lambda i: (0, 0)),
            pl.BlockSpec((n, d), lambda i: (0, 0)),
        ],
        out_specs=[
            pl.BlockSpec((1, 1, bt), lambda i: (i, 0, 0)),
            pl.BlockSpec((bt, d), lambda i: (i, 0)),
            pl.BlockSpec((1, 1), lambda i: (0, 0)),
            pl.BlockSpec((1, 1), lambda i: (0, 0)),
            pl.BlockSpec((1, 1), lambda i: (0, 0)),
        ],
        out_shape=out_shapes,
        scratch_shapes=[
            pltpu.VMEM((1, n), jnp.float32),
            pltpu.VMEM((1, 1), jnp.float32),
            pltpu.VMEM((1, 1), jnp.float32),
        ],
    )(z_flat, ct, codebook)

    quantized = q.reshape(z_e.shape)
    encoding_indices = idx3.reshape(t)
    return (quantized, com.reshape(()), emb.reshape(()),
            ent.reshape(()), encoding_indices)


# speedup vs baseline: 1.6980x; 1.0318x over previous
import functools

import jax
import jax.numpy as jnp
from jax.experimental import pallas as pl
from jax.experimental.pallas import tpu as pltpu

TEMP = 0.01
COMMIT = 0.25


def _vq_body(nsteps, total_tokens, z_ref, ct_ref, c_ref,
             idx_ref, q_ref, ent_ref, emb_ref, com_ref,
             accp_ref, accpl_ref, accsq_ref):
    i = pl.program_id(0)
    z = z_ref[...]
    ct = ct_ref[...]
    n = ct.shape[1]
    zsq = jnp.sum(z * z, axis=1, keepdims=True)
    csq = jnp.sum(ct * ct, axis=0, keepdims=True)
    dots = jax.lax.dot_general(
        z, ct, (((1,), (0,)), ((), ())),
        preferred_element_type=jnp.float32,
        precision=jax.lax.Precision.DEFAULT)
    d = zsq - 2.0 * dots + csq

    dmin = jnp.min(d, axis=1, keepdims=True)
    lane = jax.lax.broadcasted_iota(jnp.int32, d.shape, 1)
    idx = jnp.min(jnp.where(d == dmin, lane, n), axis=1)
    idx_ref[...] = idx.reshape(idx_ref.shape)

    onehot = (idx[:, None] == jax.lax.broadcasted_iota(
        jnp.int32, (1, n), 1)).astype(jnp.float32)
    q = jax.lax.dot_general(
        onehot, c_ref[...], (((1,), (0,)), ((), ())),
        preferred_element_type=jnp.float32,
        precision=jax.lax.Precision.HIGHEST)
    q_ref[...] = q
    diff = q - z
    sq = jnp.sum(diff * diff, keepdims=True).reshape(1, 1)

    aff = (-d) / TEMP
    m = jnp.max(aff, axis=1, keepdims=True)
    sh = aff - m
    e = jnp.exp(sh)
    s = jnp.sum(e, axis=1, keepdims=True)
    probs = e / s
    logp = sh - jnp.log(s)
    plsum = jnp.sum(probs * logp, keepdims=True).reshape(1, 1)
    colsum = jnp.sum(probs, axis=0, keepdims=True)

    @pl.when(i == 0)
    def _():
        accp_ref[...] = colsum
        accpl_ref[...] = plsum
        accsq_ref[...] = sq

    @pl.when(i > 0)
    def _():
        accp_ref[...] += colsum
        accpl_ref[...] += plsum
        accsq_ref[...] += sq

    @pl.when(i == nsteps - 1)
    def _():
        tt = jnp.float32(total_tokens)
        avg_probs = accp_ref[...] / tt
        avg_entropy = -jnp.sum(avg_probs * jnp.log(avg_probs + 1e-5),
                               keepdims=True).reshape(1, 1)
        sample_entropy = -(accpl_ref[...] / tt)
        ent_ref[...] = 0.1 * (sample_entropy - avg_entropy)
        msq = accsq_ref[...] / (tt * z.shape[1])
        emb_ref[...] = msq
        com_ref[...] = COMMIT * msq


def kernel(z_e, codebook):
    codebook = jnp.asarray(codebook, dtype=jnp.float32)
    n, d = codebook.shape
    z_flat = jnp.reshape(z_e, (-1, d)).astype(jnp.float32)
    t = z_flat.shape[0]
    bt = 1024
    nsteps = t // bt
    ct = codebook.T

    out_shapes = (
        jax.ShapeDtypeStruct((nsteps, 1, bt), jnp.int32),
        jax.ShapeDtypeStruct((t, d), jnp.float32),
        jax.ShapeDtypeStruct((1, 1), jnp.float32),
        jax.ShapeDtypeStruct((1, 1), jnp.float32),
        jax.ShapeDtypeStruct((1, 1), jnp.float32),
    )
    idx3, q, ent, emb, com = pl.pallas_call(
        functools.partial(_vq_body, nsteps, t),
        grid=(nsteps,),
        in_specs=[
            pl.BlockSpec((bt, d), lambda i: (i, 0)),
            pl.BlockSpec((d, n), lambda i: (0, 0)),
            pl.BlockSpec((n, d), lambda i: (0, 0)),
        ],
        out_specs=[
            pl.BlockSpec((1, 1, bt), lambda i: (i, 0, 0)),
            pl.BlockSpec((bt, d), lambda i: (i, 0)),
            pl.BlockSpec((1, 1), lambda i: (0, 0)),
            pl.BlockSpec((1, 1), lambda i: (0, 0)),
            pl.BlockSpec((1, 1), lambda i: (0, 0)),
        ],
        out_shape=out_shapes,
        scratch_shapes=[
            pltpu.VMEM((1, n), jnp.float32),
            pltpu.VMEM((1, 1), jnp.float32),
            pltpu.VMEM((1, 1), jnp.float32),
        ],
    )(z_flat, ct, codebook)

    quantized = q.reshape(z_e.shape)
    encoding_indices = idx3.reshape(t)
    return (quantized, com.reshape(()), emb.reshape(()),
            ent.reshape(()), encoding_indices)
